# baseline (speedup 1.0000x reference)
import functools

import jax
import jax.numpy as jnp
from jax import lax
from jax.experimental import pallas as pl
from jax.experimental.pallas import tpu as pltpu
from jax.experimental.pallas import tpu_sc as plsc

VOCAB = 100000
IN_DIM = 128
EMBED_DIM = 16
BATCH = 16384

NUM_CORES = 2
NUM_SUBCORES = 16
NW = NUM_CORES * NUM_SUBCORES
HALVES = 2
HALF = BATCH // HALVES
B_PER_W = HALF // NW
CHUNK = 128
N_CHUNKS = B_PER_W // CHUNK

MM_BLOCK = 4096


def _sc_gather_half(idx, table, half):
    mesh = plsc.VectorSubcoreMesh(core_axis_name="c", subcore_axis_name="s")
    base_off = half * HALF

    @functools.partial(
        pl.kernel,
        mesh=mesh,
        out_type=jax.ShapeDtypeStruct((HALF, IN_DIM), jnp.float32),
        scratch_types=[
            pltpu.VMEM((B_PER_W,), jnp.int32),
            pltpu.VMEM((B_PER_W, IN_DIM), jnp.float32),
            pltpu.SemaphoreType.DMA,
            pltpu.SemaphoreType.DMA,
        ],
    )
    def gather_k(idx_hbm, table_hbm, out_hbm, idx_v, rows_v, gsem, wsem):
        wid = lax.axis_index("s") * NUM_CORES + lax.axis_index("c")
        pltpu.sync_copy(
            idx_hbm.at[pl.ds(base_off + wid * B_PER_W, B_PER_W)], idx_v)
        gathers = [
            pltpu.async_copy(
                table_hbm.at[idx_v.at[pl.ds(j * CHUNK, CHUNK)]],
                rows_v.at[pl.ds(j * CHUNK, CHUNK)],
                gsem,
            )
            for j in range(N_CHUNKS)
        ]
        for g in gathers:
            g.wait()
        pltpu.async_copy(rows_v, out_hbm.at[pl.ds(wid * B_PER_W, B_PER_W)],
                         wsem).wait()

    return gather_k(idx, table)


def _mm_body(x_ref, w_ref, b_ref, o_ref):
    o_ref[...] = (
        lax.dot_general(
            w_ref[...], x_ref[...],
            (((1,), (1,)), ((), ())),
            preferred_element_type=jnp.float32,
        )
        + b_ref[...]
    )


def _mm_t(gathered, W_r, b2):
    return pl.pallas_call(
        _mm_body,
        grid=(HALF // MM_BLOCK,),
        in_specs=[
            pl.BlockSpec((MM_BLOCK, IN_DIM), lambda i: (i, 0)),
            pl.BlockSpec((EMBED_DIM, IN_DIM), lambda i: (0, 0)),
            pl.BlockSpec((EMBED_DIM, 1), lambda i: (0, 0)),
        ],
        out_specs=pl.BlockSpec((EMBED_DIM, MM_BLOCK), lambda i: (0, i)),
        out_shape=jax.ShapeDtypeStruct((EMBED_DIM, HALF), jnp.float32),
    )(gathered, W_r, b2)


def kernel(nodes_v, m_feature, W_r, b_r):
    idx = nodes_v.astype(jnp.int32)
    b2 = b_r.reshape(EMBED_DIM, 1)
    g0 = _sc_gather_half(idx, m_feature, 0)
    g1 = _sc_gather_half(idx, m_feature, 1)
    o0 = _mm_t(g0, W_r, b2)
    o1 = _mm_t(g1, W_r, b2)
    return jnp.concatenate([o0, o1], axis=1).T


# speedup vs baseline: 1.1930x; 1.1930x over previous
import functools

import jax
import jax.numpy as jnp
from jax import lax
from jax.experimental import pallas as pl
from jax.experimental.pallas import tpu as pltpu
from jax.experimental.pallas import tpu_sc as plsc

VOCAB = 100000
IN_DIM = 128
EMBED_DIM = 16
BATCH = 16384

NUM_CORES = 2
NUM_SUBCORES = 16
NW = NUM_CORES * NUM_SUBCORES
B_PER_W = BATCH // NW
CHUNK = 128
N_CHUNKS = B_PER_W // CHUNK

MM_BLOCK = 8192


def _sc_gather(idx, table):
    mesh = plsc.VectorSubcoreMesh(core_axis_name="c", subcore_axis_name="s")

    @functools.partial(
        pl.kernel,
        mesh=mesh,
        compiler_params=pltpu.CompilerParams(use_tc_tiling_on_sc=True),
        out_type=jax.ShapeDtypeStruct((BATCH, IN_DIM), jnp.float32),
        scratch_types=[
            pltpu.VMEM((B_PER_W,), jnp.int32),
            pltpu.VMEM((B_PER_W, IN_DIM), jnp.float32),
            pltpu.SemaphoreType.DMA,
            pltpu.SemaphoreType.DMA,
        ],
    )
    def gather_k(idx_hbm, table_hbm, out_hbm, idx_v, rows_v, gsem, wsem):
        wid = lax.axis_index("s") * NUM_CORES + lax.axis_index("c")
        pltpu.sync_copy(idx_hbm.at[pl.ds(wid * B_PER_W, B_PER_W)], idx_v)
        gathers = [
            pltpu.async_copy(
                table_hbm.at[idx_v.at[pl.ds(j * CHUNK, CHUNK)]],
                rows_v.at[pl.ds(j * CHUNK, CHUNK)],
                gsem,
            )
            for j in range(N_CHUNKS)
        ]
        for g in gathers:
            g.wait()
        pltpu.async_copy(rows_v, out_hbm.at[pl.ds(wid * B_PER_W, B_PER_W)],
                         wsem).wait()

    return gather_k(idx, table)


def _mm_body(x_ref, w_ref, b_ref, o_ref):
    o_ref[...] = (
        lax.dot_general(
            w_ref[...], x_ref[...],
            (((1,), (1,)), ((), ())),
            preferred_element_type=jnp.float32,
        )
        + b_ref[...]
    )


def _mm_t(gathered, W_r, b2):
    return pl.pallas_call(
        _mm_body,
        grid=(BATCH // MM_BLOCK,),
        in_specs=[
            pl.BlockSpec((MM_BLOCK, IN_DIM), lambda i: (i, 0)),
            pl.BlockSpec((EMBED_DIM, IN_DIM), lambda i: (0, 0)),
            pl.BlockSpec((EMBED_DIM, 1), lambda i: (0, 0)),
        ],
        out_specs=pl.BlockSpec((EMBED_DIM, MM_BLOCK), lambda i: (0, i)),
        out_shape=jax.ShapeDtypeStruct((EMBED_DIM, BATCH), jnp.float32),
    )(gathered, W_r, b2)


def kernel(nodes_v, m_feature, W_r, b_r):
    idx = nodes_v.astype(jnp.int32)
    gathered = _sc_gather(idx, m_feature)
    out_t = _mm_t(gathered, W_r, b_r.reshape(EMBED_DIM, 1))
    return out_t.T


# speedup vs baseline: 1.1951x; 1.0017x over previous
import functools

import jax
import jax.numpy as jnp
from jax import lax
from jax.experimental import pallas as pl
from jax.experimental.pallas import tpu as pltpu
from jax.experimental.pallas import tpu_sc as plsc

VOCAB = 100000
IN_DIM = 128
EMBED_DIM = 16
BATCH = 16384

NUM_CORES = 2
NUM_SUBCORES = 16
NW = NUM_CORES * NUM_SUBCORES
B_PER_W = BATCH // NW
CHUNK = 128
N_CHUNKS = B_PER_W // CHUNK

MM_BLOCK = 8192


def _sc_gather(idx, table):
    mesh = plsc.VectorSubcoreMesh(core_axis_name="c", subcore_axis_name="s")

    @functools.partial(
        pl.kernel,
        mesh=mesh,
        out_type=jax.ShapeDtypeStruct((BATCH, IN_DIM), jnp.float32),
        scratch_types=[
            pltpu.VMEM((B_PER_W,), jnp.int32),
            pltpu.VMEM((B_PER_W, IN_DIM), jnp.float32),
            pltpu.SemaphoreType.DMA,
            pltpu.SemaphoreType.DMA,
        ],
    )
    def gather_k(idx_hbm, table_hbm, out_hbm, idx_v, rows_v, gsem, wsem):
        wid = lax.axis_index("s") * NUM_CORES + lax.axis_index("c")
        pltpu.sync_copy(idx_hbm.at[pl.ds(wid * B_PER_W, B_PER_W)], idx_v)
        gathers = [
            pltpu.async_copy(
                table_hbm.at[idx_v.at[pl.ds(j * CHUNK, CHUNK)]],
                rows_v.at[pl.ds(j * CHUNK, CHUNK)],
                gsem,
            )
            for j in range(N_CHUNKS)
        ]
        for g in gathers:
            g.wait()
        pltpu.async_copy(rows_v, out_hbm.at[pl.ds(wid * B_PER_W, B_PER_W)],
                         wsem).wait()

    return gather_k(idx, table)


def _mm_body(x_ref, w_ref, b_ref, o_ref):
    o_ref[...] = (
        lax.dot_general(
            w_ref[...], x_ref[...],
            (((1,), (1,)), ((), ())),
            preferred_element_type=jnp.float32,
        )
        + b_ref[...]
    )


def _mm_t(gathered, W_r, b2):
    return pl.pallas_call(
        _mm_body,
        grid=(BATCH // MM_BLOCK,),
        in_specs=[
            pl.BlockSpec((MM_BLOCK, IN_DIM), lambda i: (i, 0)),
            pl.BlockSpec((EMBED_DIM, IN_DIM), lambda i: (0, 0)),
            pl.BlockSpec((EMBED_DIM, 1), lambda i: (0, 0)),
        ],
        out_specs=pl.BlockSpec((EMBED_DIM, MM_BLOCK), lambda i: (0, i)),
        out_shape=jax.ShapeDtypeStruct((EMBED_DIM, BATCH), jnp.float32),
    )(gathered, W_r, b2)


def kernel(nodes_v, m_feature, W_r, b_r):
    idx = nodes_v.astype(jnp.int32)
    gathered = _sc_gather(idx, m_feature)
    out_t = _mm_t(gathered, W_r, b_r.reshape(EMBED_DIM, 1))
    return out_t.T
